# Initial kernel scaffold; baseline (speedup 1.0000x reference)
import jax
import jax.numpy as jnp
from jax.experimental import pallas as pl


def kernel(x):
    raise NotImplementedError("write your pallas kernel here")


# speedup vs baseline: 9.5886x; 9.5886x over previous
import functools

import jax
import jax.numpy as jnp
from jax.experimental import pallas as pl
from jax.experimental.pallas import tpu as pltpu

_K = 64
_ROWS_PER_BLOCK = 8
_N = 32768


def _sortable_u32(x):
    u = jax.lax.bitcast_convert_type(x, jnp.uint32)
    neg = (u >> 31).astype(jnp.bool_)
    return jnp.where(neg, ~u, u | jnp.uint32(0x80000000))


def _unsortable_f32(su):
    pos = (su >> 31).astype(jnp.bool_)
    u = jnp.where(pos, su ^ jnp.uint32(0x80000000), ~su)
    return jax.lax.bitcast_convert_type(u, jnp.float32)


def _topk_mask_body(x_ref, o_ref):
    x = x_ref[...]
    su = _sortable_u32(x)
    r = x.shape[0]
    t = jnp.zeros((r, 1), dtype=jnp.uint32)
    for b in range(31, -1, -1):
        cand = t | jnp.uint32(1 << b)
        cnt = jnp.sum((su >= cand).astype(jnp.int32), axis=1, keepdims=True)
        t = jnp.where(cnt >= _K, cand, t)
    thr = _unsortable_f32(t)
    o_ref[...] = jnp.where(x >= thr, x, jnp.zeros_like(x))


@jax.jit
def kernel(x):
    m, n = x.shape
    grid = (m // _ROWS_PER_BLOCK,)
    return pl.pallas_call(
        _topk_mask_body,
        grid=grid,
        in_specs=[pl.BlockSpec((_ROWS_PER_BLOCK, n), lambda i: (i, 0))],
        out_specs=pl.BlockSpec((_ROWS_PER_BLOCK, n), lambda i: (i, 0)),
        out_shape=jax.ShapeDtypeStruct((m, n), x.dtype),
    )(x)
